# baseline (speedup 1.0000x reference)
import functools

import jax
import jax.numpy as jnp
from jax import lax
from jax.experimental import pallas as pl
from jax.experimental.pallas import tpu as pltpu
from jax.experimental.pallas import tpu_sc as plsc

D_MODEL = 64
SCALE = float(D_MODEL) ** 0.5
NUM_WORKERS = 32
STEP = 128
NBUF = 2
LANES = 16


def _emb_kernel(steps_per_w, idx_hbm, table_hbm, out_hbm,
                idx_v, gbufs, wbufs, gsems, wsems):
    nc = 2
    wid = lax.axis_index("s") * nc + lax.axis_index("c")
    per_w = steps_per_w * STEP
    base = wid * per_w
    pltpu.sync_copy(idx_hbm.at[pl.ds(base, per_w)], idx_v)

    def gather_issue(j, b):
        def row16(c, _):
            r0 = c * LANES
            chunk = idx_v[pl.ds(j * STEP + r0, LANES)]
            for l in range(LANES):
                pltpu.make_async_copy(
                    table_hbm.at[chunk[l]], gbufs[b].at[r0 + l], gsems[b]
                ).start()
            return 0

        lax.fori_loop(0, STEP // LANES, row16, 0, unroll=4)

    def gather_drain(b):
        pltpu.make_async_copy(
            out_hbm.at[pl.ds(0, STEP)], gbufs[b], gsems[b]
        ).wait()

    def write(j, b):
        return pltpu.make_async_copy(
            wbufs[b], out_hbm.at[pl.ds(base + j * STEP, STEP)], wsems[b]
        )

    gather_issue(0, 0)

    n_rounds = steps_per_w // NBUF

    def round_body(k, _):
        for b in range(NBUF):
            j = k * NBUF + b
            nb = (b + 1) % NBUF
            @pl.when(j + 1 < steps_per_w)
            def _():
                gather_issue(j + 1, nb)

            gather_drain(b)
            @pl.when(k > 0)
            def _():
                write(j - NBUF, b).wait()

            @plsc.parallel_loop(0, STEP, unroll=4)
            def _(i):
                for t in range(D_MODEL // LANES):
                    sl = pl.ds(t * LANES, LANES)
                    wbufs[b][i, sl] = gbufs[b][i, sl] * SCALE

            write(j, b).start()
        return 0

    lax.fori_loop(0, n_rounds, round_body, 0)

    for b in range(NBUF):
        write(steps_per_w - NBUF + b, b).wait()


def kernel(x, table):
    b0, b1 = x.shape
    total = b0 * b1
    n_steps = total // STEP
    steps_per_w = n_steps // NUM_WORKERS
    assert n_steps * STEP == total and steps_per_w * NUM_WORKERS == n_steps
    assert steps_per_w % NBUF == 0

    idx1d = x.reshape(total).astype(jnp.int32)

    mesh = plsc.VectorSubcoreMesh(core_axis_name="c", subcore_axis_name="s")
    out = pl.kernel(
        functools.partial(_emb_kernel, steps_per_w),
        mesh=mesh,
        out_type=jax.ShapeDtypeStruct((total, D_MODEL), jnp.float32),
        scratch_types=[
            pltpu.VMEM((steps_per_w * STEP,), jnp.int32),
            [pltpu.VMEM((STEP, D_MODEL), jnp.float32) for _ in range(NBUF)],
            [pltpu.VMEM((STEP, D_MODEL), jnp.float32) for _ in range(NBUF)],
            [pltpu.SemaphoreType.DMA for _ in range(NBUF)],
            [pltpu.SemaphoreType.DMA for _ in range(NBUF)],
        ],
    )(idx1d, table)
    return out.reshape(b0, b1, D_MODEL)
